# baseline (speedup 1.0000x reference)
import functools

import jax
import jax.numpy as jnp
from jax.experimental import pallas as pl
from jax.experimental.pallas import tpu as pltpu
from jax.experimental.pallas import tpu_sc as plsc

_KT = 1024
_NT = 1024


def _prep_body(z_ref, pwt_ref, pb_ref, e_ref,
               f2x_out, fbf_out, f2_out, e2_out):
    f = jnp.dot(z_ref[...], pwt_ref[...],
                preferred_element_type=jnp.float32) + pb_ref[...]
    f2x = f + f
    f2x_out[...] = f2x
    fbf_out[...] = f2x.astype(jnp.bfloat16)
    f2_out[...] = jnp.sum(f * f, axis=1, keepdims=True)
    e = e_ref[...]
    e2_out[...] = jnp.sum(e * e, axis=1, keepdims=True)


def _qa_body(kt_last, n_total, f2x_ref, f2_ref, embt_ref, e2_ref,
             lse_out, ind_out, diff_out, s_s, bm_s, bi_s, acc_s):
    nb = pl.program_id(0)
    kt = pl.program_id(1)
    nbt = pl.num_programs(0)
    nt = s_s.shape[0]

    @pl.when(kt == 0)
    def _init():
        s_s[...] = jnp.zeros((nt, 1), jnp.float32)
        bm_s[...] = jnp.full((nt, 1), jnp.inf, jnp.float32)
        bi_s[...] = jnp.zeros((nt, 1), jnp.float32)

    e = embt_ref[:, pl.ds(kt * _KT, _KT)]
    m2 = jnp.dot(f2x_ref[...], e, preferred_element_type=jnp.float32)
    dist = (f2_ref[...] - m2) + e2_ref[...]

    tmin = jnp.min(dist, axis=1, keepdims=True)
    cols = jax.lax.broadcasted_iota(jnp.int32, dist.shape, 1)
    big = jnp.int32(2147480000)
    idx = jnp.min(jnp.where(dist == tmin, cols, big), axis=1, keepdims=True)
    idx = (idx + kt * _KT).astype(jnp.float32)

    bm_old = bm_s[...]
    bm_new = jnp.minimum(bm_old, tmin)
    s_s[...] = (s_s[...] * jnp.exp(bm_new - bm_old)
                + jnp.sum(jnp.exp(bm_new - dist), axis=1, keepdims=True))
    bm_s[...] = bm_new
    flip = tmin < bm_old
    bi_s[...] = jnp.where(flip, idx, bi_s[...])

    @pl.when(kt == kt_last)
    def _fin():
        lse_out[...] = jnp.log(s_s[...]) - bm_s[...]
        ind_out[...] = bi_s[...].astype(jnp.int32)
        part = jnp.sum(bm_s[...]).reshape(1, 1)
        tot = jnp.where(nb == 0, part, acc_s[...] + part)
        acc_s[...] = tot

        @pl.when(nb == nbt - 1)
        def _done():
            diff_out[...] = tot * jnp.float32(12.5 / n_total)


def _qb_body(embt_ref, f_ref, f2_ref, lse_ref, e2_ref, lp_out, ebf_s):
    nb = pl.program_id(0)
    kt = pl.program_id(1)

    @pl.when(nb == 0)
    def _stage():
        ebf_s[:, pl.ds(kt * _KT, _KT)] = (
            embt_ref[:, pl.ds(kt * _KT, _KT)].astype(jnp.bfloat16))

    e = ebf_s[:, pl.ds(kt * _KT, _KT)]
    m2 = jnp.dot(f_ref[...], e, preferred_element_type=jnp.float32)
    lp_out[...] = ((m2 - f2_ref[...]) - e2_ref[...]) - lse_ref[...]


def _gather_rows(ind2, embed_w):
    nrow, lanes = ind2.shape
    n = nrow * lanes
    k, d = embed_w.shape
    nw = 32
    bpw = n // nw
    chunks = bpw // lanes
    mesh = plsc.VectorSubcoreMesh(core_axis_name="c", subcore_axis_name="s")

    @functools.partial(
        pl.kernel,
        out_type=jax.ShapeDtypeStruct((n, d), jnp.float32),
        mesh=mesh,
        scratch_types=[
            pltpu.VMEM((chunks, lanes), jnp.int32),
            pltpu.VMEM((bpw, d), jnp.float32),
            pltpu.SemaphoreType.DMA,
        ],
    )
    def gk(idx_hbm, tab_hbm, out_hbm, idx_v, rows_v, sem):
        wid = jax.lax.axis_index("s") * 2 + jax.lax.axis_index("c")
        pltpu.sync_copy(idx_hbm.at[pl.ds(wid * chunks, chunks)], idx_v)
        cps = [
            pltpu.async_copy(tab_hbm.at[idx_v.at[j]],
                             rows_v.at[pl.ds(j * lanes, lanes)], sem)
            for j in range(chunks)
        ]
        for cp in cps:
            cp.wait()
        pltpu.sync_copy(rows_v, out_hbm.at[pl.ds(wid * bpw, bpw)])

    return gk(ind2, embed_w)


def kernel(z, proj_w, proj_b, embed_w):
    bz, c, h, w = z.shape
    d = proj_w.shape[0]
    k = embed_w.shape[0]
    n = bz * h * w
    nk = k // _KT
    nb = n // _NT

    z_rows = z.transpose(0, 2, 3, 1).reshape(n, c)
    pwt = proj_w.T
    pb_row = proj_b.reshape(1, d)
    embt = embed_w.T

    f2x, fbf, f2col, e2col = pl.pallas_call(
        _prep_body,
        grid=(nb,),
        in_specs=[
            pl.BlockSpec((_NT, c), lambda t: (t, 0)),
            pl.BlockSpec((c, d), lambda t: (0, 0)),
            pl.BlockSpec((1, d), lambda t: (0, 0)),
            pl.BlockSpec((_KT, d), lambda t: (t, 0)),
        ],
        out_specs=[
            pl.BlockSpec((_NT, d), lambda t: (t, 0)),
            pl.BlockSpec((_NT, d), lambda t: (t, 0)),
            pl.BlockSpec((_NT, 1), lambda t: (t, 0)),
            pl.BlockSpec((_KT, 1), lambda t: (t, 0)),
        ],
        out_shape=[
            jax.ShapeDtypeStruct((n, d), jnp.float32),
            jax.ShapeDtypeStruct((n, d), jnp.bfloat16),
            jax.ShapeDtypeStruct((n, 1), jnp.float32),
            jax.ShapeDtypeStruct((k, 1), jnp.float32),
        ],
    )(z_rows, pwt, pb_row, embed_w)
    e2row = e2col.reshape(1, k)

    lsecol, indcol, diffo = pl.pallas_call(
        functools.partial(_qa_body, nk - 1, n * d),
        grid=(nb, nk),
        in_specs=[
            pl.BlockSpec((_NT, d), lambda b, t: (b, 0)),
            pl.BlockSpec((_NT, 1), lambda b, t: (b, 0)),
            pl.BlockSpec((d, k), lambda b, t: (0, 0)),
            pl.BlockSpec((1, _KT), lambda b, t: (0, t)),
        ],
        out_specs=[
            pl.BlockSpec((_NT, 1), lambda b, t: (b, 0)),
            pl.BlockSpec((_NT, 1), lambda b, t: (b, 0)),
            pl.BlockSpec((1, 1), lambda b, t: (0, 0)),
        ],
        out_shape=[
            jax.ShapeDtypeStruct((n, 1), jnp.float32),
            jax.ShapeDtypeStruct((n, 1), jnp.int32),
            jax.ShapeDtypeStruct((1, 1), jnp.float32),
        ],
        scratch_shapes=[
            pltpu.VMEM((_NT, 1), jnp.float32),
            pltpu.VMEM((_NT, 1), jnp.float32),
            pltpu.VMEM((_NT, 1), jnp.float32),
            pltpu.VMEM((1, 1), jnp.float32),
        ],
    )(f2x, f2col, embt, e2row)

    ind_flat = indcol.reshape(n)
    zq_flat = _gather_rows(ind_flat.reshape(n // 128, 128), embed_w)

    lp = pl.pallas_call(
        _qb_body,
        grid=(nb, nk),
        in_specs=[
            pl.BlockSpec((d, k), lambda b, t: (0, 0)),
            pl.BlockSpec((_NT, d), lambda b, t: (b, 0)),
            pl.BlockSpec((_NT, 1), lambda b, t: (b, 0)),
            pl.BlockSpec((_NT, 1), lambda b, t: (b, 0)),
            pl.BlockSpec((1, _KT), lambda b, t: (0, t)),
        ],
        out_specs=pl.BlockSpec((_NT, _KT), lambda b, t: (b, t)),
        out_shape=jax.ShapeDtypeStruct((n, k), jnp.float32),
        scratch_shapes=[pltpu.VMEM((d, k), jnp.bfloat16)],
    )(embt, fbf, f2col, lsecol, e2row)

    z_q = zq_flat.reshape(bz, h, w, d).transpose(0, 3, 1, 2)
    log_priors = lp.reshape(bz, h, w, k).transpose(0, 3, 1, 2)
    ind = ind_flat.reshape(bz, h, w)
    diff = diffo.reshape(())
    return (z_q, diff, ind, log_priors)


# speedup vs baseline: 1.0292x; 1.0292x over previous
import functools

import jax
import jax.numpy as jnp
from jax.experimental import pallas as pl
from jax.experimental.pallas import tpu as pltpu
from jax.experimental.pallas import tpu_sc as plsc

_KT = 1024
_NT = 1024


def _qa_body(kt_last, n_total, z_ref, pwt_ref, pb_ref, embt_ref, eo_ref,
             fbf_out, f2_out, lse_out, ind_out, diff_out,
             f2x_s, f2_s, e2_s, s_s, bm_s, bi_s, acc_s):
    nb = pl.program_id(0)
    kt = pl.program_id(1)
    nbt = pl.num_programs(0)
    nt = f2x_s.shape[0]

    @pl.when(kt == 0)
    def _init():
        f = jnp.dot(z_ref[...], pwt_ref[...],
                    preferred_element_type=jnp.float32) + pb_ref[...]
        f2x = f + f
        f2x_s[...] = f2x
        fbf_out[...] = f2x.astype(jnp.bfloat16)
        f2 = jnp.sum(f * f, axis=1, keepdims=True)
        f2_s[...] = f2
        f2_out[...] = f2
        s_s[...] = jnp.zeros((nt, 1), jnp.float32)
        bm_s[...] = jnp.full((nt, 1), jnp.inf, jnp.float32)
        bi_s[...] = jnp.zeros((nt, 1), jnp.float32)

    @pl.when(nb == 0)
    def _norms():
        eo = eo_ref[...]
        e2c = jnp.sum(eo * eo, axis=1, keepdims=True)
        e2_s[:, pl.ds(kt * _KT, _KT)] = e2c.reshape(1, _KT)

    e = embt_ref[:, pl.ds(kt * _KT, _KT)]
    m2 = jnp.dot(f2x_s[...], e, preferred_element_type=jnp.float32)
    dist = (f2_s[...] - m2) + e2_s[:, pl.ds(kt * _KT, _KT)]

    tmin = jnp.min(dist, axis=1, keepdims=True)
    cols = jax.lax.broadcasted_iota(jnp.int32, dist.shape, 1)
    big = jnp.int32(2147480000)
    idx = jnp.min(jnp.where(dist == tmin, cols, big), axis=1, keepdims=True)
    idx = (idx + kt * _KT).astype(jnp.float32)

    bm_old = bm_s[...]
    bm_new = jnp.minimum(bm_old, tmin)
    s_s[...] = (s_s[...] * jnp.exp(bm_new - bm_old)
                + jnp.sum(jnp.exp(bm_new - dist), axis=1, keepdims=True))
    bm_s[...] = bm_new
    flip = tmin < bm_old
    bi_s[...] = jnp.where(flip, idx, bi_s[...])

    @pl.when(kt == kt_last)
    def _fin():
        lse_out[...] = jnp.log(s_s[...]) - bm_s[...]
        ind_out[...] = bi_s[...].astype(jnp.int32).reshape(ind_out.shape)
        part = jnp.sum(bm_s[...]).reshape(1, 1)
        tot = jnp.where(nb == 0, part, acc_s[...] + part)
        acc_s[...] = tot

        @pl.when(nb == nbt - 1)
        def _done():
            diff_out[...] = tot * jnp.float32(12.5 / n_total)


def _qb_body(embt_ref, f_ref, f2_ref, lse_ref, lp_out, ebf_s, e2_s):
    nb = pl.program_id(0)
    kt = pl.program_id(1)

    @pl.when(nb == 0)
    def _stage():
        es = embt_ref[:, pl.ds(kt * _KT, _KT)]
        ebf_s[:, pl.ds(kt * _KT, _KT)] = es.astype(jnp.bfloat16)
        e2_s[:, pl.ds(kt * _KT, _KT)] = jnp.sum(es * es, axis=0, keepdims=True)

    e = ebf_s[:, pl.ds(kt * _KT, _KT)]
    m2 = jnp.dot(f_ref[...], e, preferred_element_type=jnp.float32)
    lp_out[...] = ((m2 - f2_ref[...])
                   - e2_s[:, pl.ds(kt * _KT, _KT)]) - lse_ref[...]


def _gather_rows(ind2, embed_w):
    nrow, lanes = ind2.shape
    n = nrow * lanes
    k, d = embed_w.shape
    nw = 32
    bpw = n // nw
    chunks = bpw // lanes
    mesh = plsc.VectorSubcoreMesh(core_axis_name="c", subcore_axis_name="s")

    @functools.partial(
        pl.kernel,
        out_type=jax.ShapeDtypeStruct((n, d), jnp.float32),
        mesh=mesh,
        scratch_types=[
            pltpu.VMEM((chunks, lanes), jnp.int32),
            pltpu.VMEM((bpw, d), jnp.float32),
            pltpu.SemaphoreType.DMA,
        ],
    )
    def gk(idx_hbm, tab_hbm, out_hbm, idx_v, rows_v, sem):
        wid = jax.lax.axis_index("s") * 2 + jax.lax.axis_index("c")
        pltpu.sync_copy(idx_hbm.at[pl.ds(wid * chunks, chunks)], idx_v)
        cps = [
            pltpu.async_copy(tab_hbm.at[idx_v.at[j]],
                             rows_v.at[pl.ds(j * lanes, lanes)], sem)
            for j in range(chunks)
        ]
        for cp in cps:
            cp.wait()
        pltpu.sync_copy(rows_v, out_hbm.at[pl.ds(wid * bpw, bpw)])

    return gk(ind2, embed_w)


def kernel(z, proj_w, proj_b, embed_w):
    bz, c, h, w = z.shape
    d = proj_w.shape[0]
    k = embed_w.shape[0]
    n = bz * h * w
    nk = k // _KT
    nb = n // _NT

    z_rows = z.transpose(0, 2, 3, 1).reshape(n, c)
    pwt = proj_w.T
    pb_row = proj_b.reshape(1, d)
    embt = embed_w.T

    fbf, f2col, lsecol, ind64, diffo = pl.pallas_call(
        functools.partial(_qa_body, nk - 1, n * d),
        grid=(nb, nk),
        in_specs=[
            pl.BlockSpec((_NT, c), lambda b, t: (b, 0)),
            pl.BlockSpec((c, d), lambda b, t: (0, 0)),
            pl.BlockSpec((1, d), lambda b, t: (0, 0)),
            pl.BlockSpec((d, k), lambda b, t: (0, 0)),
            pl.BlockSpec((_KT, d), lambda b, t: (t, 0)),
        ],
        out_specs=[
            pl.BlockSpec((_NT, d), lambda b, t: (b, 0)),
            pl.BlockSpec((_NT, 1), lambda b, t: (b, 0)),
            pl.BlockSpec((_NT, 1), lambda b, t: (b, 0)),
            pl.BlockSpec((_NT // 128, 128), lambda b, t: (b, 0)),
            pl.BlockSpec((1, 1), lambda b, t: (0, 0)),
        ],
        out_shape=[
            jax.ShapeDtypeStruct((n, d), jnp.bfloat16),
            jax.ShapeDtypeStruct((n, 1), jnp.float32),
            jax.ShapeDtypeStruct((n, 1), jnp.float32),
            jax.ShapeDtypeStruct((n // 128, 128), jnp.int32),
            jax.ShapeDtypeStruct((1, 1), jnp.float32),
        ],
        scratch_shapes=[
            pltpu.VMEM((_NT, d), jnp.float32),
            pltpu.VMEM((_NT, 1), jnp.float32),
            pltpu.VMEM((1, k), jnp.float32),
            pltpu.VMEM((_NT, 1), jnp.float32),
            pltpu.VMEM((_NT, 1), jnp.float32),
            pltpu.VMEM((_NT, 1), jnp.float32),
            pltpu.VMEM((1, 1), jnp.float32),
        ],
    )(z_rows, pwt, pb_row, embt, embed_w)

    zq_flat = _gather_rows(ind64, embed_w)

    lp = pl.pallas_call(
        _qb_body,
        grid=(nb, nk),
        in_specs=[
            pl.BlockSpec((d, k), lambda b, t: (0, 0)),
            pl.BlockSpec((_NT, d), lambda b, t: (b, 0)),
            pl.BlockSpec((_NT, 1), lambda b, t: (b, 0)),
            pl.BlockSpec((_NT, 1), lambda b, t: (b, 0)),
        ],
        out_specs=pl.BlockSpec((_NT, _KT), lambda b, t: (b, t)),
        out_shape=jax.ShapeDtypeStruct((n, k), jnp.float32),
        scratch_shapes=[pltpu.VMEM((d, k), jnp.bfloat16),
                        pltpu.VMEM((1, k), jnp.float32)],
    )(embt, fbf, f2col, lsecol)

    z_q = zq_flat.reshape(bz, h, w, d).transpose(0, 3, 1, 2)
    log_priors = lp.reshape(bz, h, w, k).transpose(0, 3, 1, 2)
    ind = ind64.reshape(bz, h, w)
    diff = diffo.reshape(())
    return (z_q, diff, ind, log_priors)


# speedup vs baseline: 1.0424x; 1.0129x over previous
import functools

import jax
import jax.numpy as jnp
from jax.experimental import pallas as pl
from jax.experimental.pallas import tpu as pltpu
from jax.experimental.pallas import tpu_sc as plsc

_KT = 1024
_NT = 1024


def _qa_body(kt_last, n_total, z_ref, pwt_ref, pb_ref, embt_ref, eo_ref,
             fbf_out, f2_out, lse_out, ind_out, diff_out,
             f2x_s, f2_s, e2_s, s_s, bm_s, bi_s, acc_s):
    nb = pl.program_id(0)
    kt = pl.program_id(1)
    nbt = pl.num_programs(0)
    nt = f2x_s.shape[0]

    @pl.when(kt == 0)
    def _init():
        f = jnp.dot(z_ref[...], pwt_ref[...],
                    preferred_element_type=jnp.float32) + pb_ref[...]
        f2x = f + f
        f2x_s[...] = f2x
        fbf_out[...] = f2x.astype(jnp.bfloat16)
        f2 = jnp.sum(f * f, axis=1, keepdims=True)
        f2_s[...] = f2
        f2_out[...] = f2
        s_s[...] = jnp.zeros((nt, 1), jnp.float32)
        bm_s[...] = jnp.full((nt, 1), jnp.inf, jnp.float32)
        bi_s[...] = jnp.zeros((nt, 1), jnp.float32)

    @pl.when(nb == 0)
    def _norms():
        eo = eo_ref[...]
        e2c = jnp.sum(eo * eo, axis=1, keepdims=True)
        e2_s[:, pl.ds(kt * _KT, _KT)] = e2c.reshape(1, _KT)

    e = embt_ref[:, pl.ds(kt * _KT, _KT)]
    m2 = jnp.dot(f2x_s[...], e, preferred_element_type=jnp.float32)
    dist = (f2_s[...] - m2) + e2_s[:, pl.ds(kt * _KT, _KT)]

    tmin = jnp.min(dist, axis=1, keepdims=True)
    cols = jax.lax.broadcasted_iota(jnp.int32, dist.shape, 1)
    big = jnp.int32(2147480000)
    idx = jnp.min(jnp.where(dist == tmin, cols, big), axis=1, keepdims=True)
    idx = (idx + kt * _KT).astype(jnp.float32)

    bm_old = bm_s[...]
    bm_new = jnp.minimum(bm_old, tmin)
    s_s[...] = (s_s[...] * jnp.exp(bm_new - bm_old)
                + jnp.sum(jnp.exp(bm_new - dist), axis=1, keepdims=True))
    bm_s[...] = bm_new
    flip = tmin < bm_old
    bi_s[...] = jnp.where(flip, idx, bi_s[...])

    @pl.when(kt == kt_last)
    def _fin():
        lse_out[...] = jnp.log(s_s[...]) - bm_s[...]
        ind_out[...] = bi_s[...].astype(jnp.int32).reshape(ind_out.shape)
        part = jnp.sum(bm_s[...]).reshape(1, 1)
        tot = jnp.where(nb == 0, part, acc_s[...] + part)
        acc_s[...] = tot

        @pl.when(nb == nbt - 1)
        def _done():
            diff_out[...] = tot * jnp.float32(12.5 / n_total)


_KTB = 2048


def _qb_body(embt_ref, f_ref, f2_ref, lse_ref, lp_out, ebf_s, e2_s):
    nb = pl.program_id(0)
    kt = pl.program_id(1)

    @pl.when(nb == 0)
    def _stage():
        es = embt_ref[:, pl.ds(kt * _KTB, _KTB)]
        ebf_s[:, pl.ds(kt * _KTB, _KTB)] = es.astype(jnp.bfloat16)
        e2_s[:, pl.ds(kt * _KTB, _KTB)] = jnp.sum(es * es, axis=0,
                                                  keepdims=True)

    e = ebf_s[:, pl.ds(kt * _KTB, _KTB)]
    m2 = jnp.dot(f_ref[...], e, preferred_element_type=jnp.float32)
    c1 = f2_ref[...] + lse_ref[...]
    lp_out[...] = (m2 - e2_s[:, pl.ds(kt * _KTB, _KTB)]) - c1


def _gather_rows(ind2, embed_w):
    nrow, lanes = ind2.shape
    n = nrow * lanes
    k, d = embed_w.shape
    nw = 32
    bpw = n // nw
    chunks = bpw // lanes
    mesh = plsc.VectorSubcoreMesh(core_axis_name="c", subcore_axis_name="s")

    @functools.partial(
        pl.kernel,
        out_type=jax.ShapeDtypeStruct((n, d), jnp.float32),
        mesh=mesh,
        scratch_types=[
            pltpu.VMEM((chunks, lanes), jnp.int32),
            pltpu.VMEM((bpw, d), jnp.float32),
            pltpu.SemaphoreType.DMA,
        ],
    )
    def gk(idx_hbm, tab_hbm, out_hbm, idx_v, rows_v, sem):
        wid = jax.lax.axis_index("s") * 2 + jax.lax.axis_index("c")
        pltpu.sync_copy(idx_hbm.at[pl.ds(wid * chunks, chunks)], idx_v)
        cps = [
            pltpu.async_copy(tab_hbm.at[idx_v.at[j]],
                             rows_v.at[pl.ds(j * lanes, lanes)], sem)
            for j in range(chunks)
        ]
        for cp in cps:
            cp.wait()
        pltpu.sync_copy(rows_v, out_hbm.at[pl.ds(wid * bpw, bpw)])

    return gk(ind2, embed_w)


def kernel(z, proj_w, proj_b, embed_w):
    bz, c, h, w = z.shape
    d = proj_w.shape[0]
    k = embed_w.shape[0]
    n = bz * h * w
    nk = k // _KT
    nb = n // _NT

    z_rows = z.transpose(0, 2, 3, 1).reshape(n, c)
    pwt = proj_w.T
    pb_row = proj_b.reshape(1, d)
    embt = embed_w.T

    fbf, f2col, lsecol, ind64, diffo = pl.pallas_call(
        functools.partial(_qa_body, nk - 1, n * d),
        grid=(nb, nk),
        in_specs=[
            pl.BlockSpec((_NT, c), lambda b, t: (b, 0)),
            pl.BlockSpec((c, d), lambda b, t: (0, 0)),
            pl.BlockSpec((1, d), lambda b, t: (0, 0)),
            pl.BlockSpec((d, k), lambda b, t: (0, 0)),
            pl.BlockSpec((_KT, d), lambda b, t: (t, 0)),
        ],
        out_specs=[
            pl.BlockSpec((_NT, d), lambda b, t: (b, 0)),
            pl.BlockSpec((_NT, 1), lambda b, t: (b, 0)),
            pl.BlockSpec((_NT, 1), lambda b, t: (b, 0)),
            pl.BlockSpec((_NT // 128, 128), lambda b, t: (b, 0)),
            pl.BlockSpec((1, 1), lambda b, t: (0, 0)),
        ],
        out_shape=[
            jax.ShapeDtypeStruct((n, d), jnp.bfloat16),
            jax.ShapeDtypeStruct((n, 1), jnp.float32),
            jax.ShapeDtypeStruct((n, 1), jnp.float32),
            jax.ShapeDtypeStruct((n // 128, 128), jnp.int32),
            jax.ShapeDtypeStruct((1, 1), jnp.float32),
        ],
        scratch_shapes=[
            pltpu.VMEM((_NT, d), jnp.float32),
            pltpu.VMEM((_NT, 1), jnp.float32),
            pltpu.VMEM((1, k), jnp.float32),
            pltpu.VMEM((_NT, 1), jnp.float32),
            pltpu.VMEM((_NT, 1), jnp.float32),
            pltpu.VMEM((_NT, 1), jnp.float32),
            pltpu.VMEM((1, 1), jnp.float32),
        ],
    )(z_rows, pwt, pb_row, embt, embed_w)

    zq_flat = _gather_rows(ind64, embed_w)

    lp = pl.pallas_call(
        _qb_body,
        grid=(nb, k // _KTB),
        in_specs=[
            pl.BlockSpec((d, k), lambda b, t: (0, 0)),
            pl.BlockSpec((_NT, d), lambda b, t: (b, 0)),
            pl.BlockSpec((_NT, 1), lambda b, t: (b, 0)),
            pl.BlockSpec((_NT, 1), lambda b, t: (b, 0)),
        ],
        out_specs=pl.BlockSpec((_NT, _KTB), lambda b, t: (b, t)),
        out_shape=jax.ShapeDtypeStruct((n, k), jnp.float32),
        scratch_shapes=[pltpu.VMEM((d, k), jnp.bfloat16),
                        pltpu.VMEM((1, k), jnp.float32)],
    )(embt, fbf, f2col, lsecol)

    z_q = zq_flat.reshape(bz, h, w, d).transpose(0, 3, 1, 2)
    log_priors = lp.reshape(bz, h, w, k).transpose(0, 3, 1, 2)
    ind = ind64.reshape(bz, h, w)
    diff = diffo.reshape(())
    return (z_q, diff, ind, log_priors)


# speedup vs baseline: 1.0652x; 1.0218x over previous
import functools

import jax
import jax.numpy as jnp
from jax.experimental import pallas as pl
from jax.experimental.pallas import tpu as pltpu
from jax.experimental.pallas import tpu_sc as plsc

_KT = 1024
_NT = 1024


def _qa_body(kt_last, n_total, z_ref, pwt_ref, pb_ref, embt_ref, eo_ref,
             fbf_out, f2_out, bm_out, ind_out, diff_out,
             f2x_s, f2_s, e2_s, bm_s, bi_s, acc_s):
    nb = pl.program_id(0)
    kt = pl.program_id(1)
    nbt = pl.num_programs(0)
    nt = f2x_s.shape[0]

    @pl.when(kt == 0)
    def _init():
        f = jnp.dot(z_ref[...], pwt_ref[...],
                    preferred_element_type=jnp.float32) + pb_ref[...]
        f2x = f + f
        f2x_s[...] = f2x
        fbf_out[...] = f2x.astype(jnp.bfloat16)
        f2 = jnp.sum(f * f, axis=1, keepdims=True)
        f2_s[...] = f2
        f2_out[...] = f2
        bm_s[...] = jnp.full((nt, 1), jnp.inf, jnp.float32)
        bi_s[...] = jnp.zeros((nt, 1), jnp.float32)

    @pl.when(nb == 0)
    def _norms():
        eo = eo_ref[...]
        e2c = jnp.sum(eo * eo, axis=1, keepdims=True)
        e2_s[:, pl.ds(kt * _KT, _KT)] = e2c.reshape(1, _KT)

    e = embt_ref[:, pl.ds(kt * _KT, _KT)]
    m2 = jnp.dot(f2x_s[...], e, preferred_element_type=jnp.float32)
    dist = (f2_s[...] - m2) + e2_s[:, pl.ds(kt * _KT, _KT)]

    tmin = jnp.min(dist, axis=1, keepdims=True)
    cols = jax.lax.broadcasted_iota(jnp.int32, dist.shape, 1)
    big = jnp.int32(2147480000)
    idx = jnp.min(jnp.where(dist == tmin, cols, big), axis=1, keepdims=True)
    idx = (idx + kt * _KT).astype(jnp.float32)

    bm_old = bm_s[...]
    bm_s[...] = jnp.minimum(bm_old, tmin)
    flip = tmin < bm_old
    bi_s[...] = jnp.where(flip, idx, bi_s[...])

    @pl.when(kt == kt_last)
    def _fin():
        bm_out[...] = bm_s[...]
        ind_out[...] = bi_s[...].astype(jnp.int32).reshape(ind_out.shape)
        part = jnp.sum(bm_s[...]).reshape(1, 1)
        tot = jnp.where(nb == 0, part, acc_s[...] + part)
        acc_s[...] = tot

        @pl.when(nb == nbt - 1)
        def _done():
            diff_out[...] = tot * jnp.float32(12.5 / n_total)


_KTB = 2048


def _qb_body(embt_ref, f_ref, f2_ref, bm_ref, lp_out, ebf_s, e2_s, s_s, c_s):
    nb = pl.program_id(0)
    p = pl.program_id(1)
    kt = pl.program_id(2)
    nt = s_s.shape[0]

    @pl.when((nb == 0) & (p == 0))
    def _stage():
        es = embt_ref[:, pl.ds(kt * _KTB, _KTB)]
        ebf_s[:, pl.ds(kt * _KTB, _KTB)] = es.astype(jnp.bfloat16)
        e2_s[:, pl.ds(kt * _KTB, _KTB)] = jnp.sum(es * es, axis=0,
                                                  keepdims=True)

    e = ebf_s[:, pl.ds(kt * _KTB, _KTB)]
    m2 = jnp.dot(f_ref[...], e, preferred_element_type=jnp.float32)
    me = m2 - e2_s[:, pl.ds(kt * _KTB, _KTB)]

    @pl.when(p == 0)
    def _accum():
        @pl.when(kt == 0)
        def _z():
            s_s[...] = jnp.zeros((nt, 1), jnp.float32)
        cb = bm_ref[...] - f2_ref[...]
        s_s[...] = s_s[...] + jnp.sum(jnp.exp(me + cb), axis=1, keepdims=True)

    @pl.when(p == 1)
    def _write():
        @pl.when(kt == 0)
        def _c():
            c_s[...] = (f2_ref[...] - bm_ref[...]) + jnp.log(s_s[...])
        lp_out[...] = me - c_s[...]


def _gather_rows(ind2, embed_w):
    nrow, lanes = ind2.shape
    n = nrow * lanes
    k, d = embed_w.shape
    nw = 32
    bpw = n // nw
    chunks = bpw // lanes
    mesh = plsc.VectorSubcoreMesh(core_axis_name="c", subcore_axis_name="s")

    @functools.partial(
        pl.kernel,
        out_type=jax.ShapeDtypeStruct((n, d), jnp.float32),
        mesh=mesh,
        scratch_types=[
            pltpu.VMEM((chunks, lanes), jnp.int32),
            pltpu.VMEM((bpw, d), jnp.float32),
            pltpu.SemaphoreType.DMA,
        ],
    )
    def gk(idx_hbm, tab_hbm, out_hbm, idx_v, rows_v, sem):
        wid = jax.lax.axis_index("s") * 2 + jax.lax.axis_index("c")
        pltpu.sync_copy(idx_hbm.at[pl.ds(wid * chunks, chunks)], idx_v)
        cps = [
            pltpu.async_copy(tab_hbm.at[idx_v.at[j]],
                             rows_v.at[pl.ds(j * lanes, lanes)], sem)
            for j in range(chunks)
        ]
        for cp in cps:
            cp.wait()
        pltpu.sync_copy(rows_v, out_hbm.at[pl.ds(wid * bpw, bpw)])

    return gk(ind2, embed_w)


def kernel(z, proj_w, proj_b, embed_w):
    bz, c, h, w = z.shape
    d = proj_w.shape[0]
    k = embed_w.shape[0]
    n = bz * h * w
    nk = k // _KT
    nb = n // _NT

    z_rows = z.transpose(0, 2, 3, 1).reshape(n, c)
    pwt = proj_w.T
    pb_row = proj_b.reshape(1, d)
    embt = embed_w.T

    fbf, f2col, bmcol, ind64, diffo = pl.pallas_call(
        functools.partial(_qa_body, nk - 1, n * d),
        grid=(nb, nk),
        in_specs=[
            pl.BlockSpec((_NT, c), lambda b, t: (b, 0)),
            pl.BlockSpec((c, d), lambda b, t: (0, 0)),
            pl.BlockSpec((1, d), lambda b, t: (0, 0)),
            pl.BlockSpec((d, k), lambda b, t: (0, 0)),
            pl.BlockSpec((_KT, d), lambda b, t: (t, 0)),
        ],
        out_specs=[
            pl.BlockSpec((_NT, d), lambda b, t: (b, 0)),
            pl.BlockSpec((_NT, 1), lambda b, t: (b, 0)),
            pl.BlockSpec((_NT, 1), lambda b, t: (b, 0)),
            pl.BlockSpec((_NT // 128, 128), lambda b, t: (b, 0)),
            pl.BlockSpec((1, 1), lambda b, t: (0, 0)),
        ],
        out_shape=[
            jax.ShapeDtypeStruct((n, d), jnp.bfloat16),
            jax.ShapeDtypeStruct((n, 1), jnp.float32),
            jax.ShapeDtypeStruct((n, 1), jnp.float32),
            jax.ShapeDtypeStruct((n // 128, 128), jnp.int32),
            jax.ShapeDtypeStruct((1, 1), jnp.float32),
        ],
        scratch_shapes=[
            pltpu.VMEM((_NT, d), jnp.float32),
            pltpu.VMEM((_NT, 1), jnp.float32),
            pltpu.VMEM((1, k), jnp.float32),
            pltpu.VMEM((_NT, 1), jnp.float32),
            pltpu.VMEM((_NT, 1), jnp.float32),
            pltpu.VMEM((1, 1), jnp.float32),
        ],
    )(z_rows, pwt, pb_row, embt, embed_w)

    zq_flat = _gather_rows(ind64, embed_w)

    lp = pl.pallas_call(
        _qb_body,
        grid=(nb, 2, k // _KTB),
        in_specs=[
            pl.BlockSpec((d, k), lambda b, p, t: (0, 0)),
            pl.BlockSpec((_NT, d), lambda b, p, t: (b, 0)),
            pl.BlockSpec((_NT, 1), lambda b, p, t: (b, 0)),
            pl.BlockSpec((_NT, 1), lambda b, p, t: (b, 0)),
        ],
        out_specs=pl.BlockSpec((_NT, _KTB), lambda b, p, t: (b, t * p)),
        out_shape=jax.ShapeDtypeStruct((n, k), jnp.float32),
        scratch_shapes=[pltpu.VMEM((d, k), jnp.bfloat16),
                        pltpu.VMEM((1, k), jnp.float32),
                        pltpu.VMEM((_NT, 1), jnp.float32),
                        pltpu.VMEM((_NT, 1), jnp.float32)],
    )(embt, fbf, f2col, bmcol)

    z_q = zq_flat.reshape(bz, h, w, d).transpose(0, 3, 1, 2)
    log_priors = lp.reshape(bz, h, w, k).transpose(0, 3, 1, 2)
    ind = ind64.reshape(bz, h, w)
    diff = diffo.reshape(())
    return (z_q, diff, ind, log_priors)


# speedup vs baseline: 1.0655x; 1.0003x over previous
import functools

import jax
import jax.numpy as jnp
from jax.experimental import pallas as pl
from jax.experimental.pallas import tpu as pltpu
from jax.experimental.pallas import tpu_sc as plsc

_KT = 1024
_NT = 1024


def _qa_body(kt_last, n_total, z_ref, pwt_ref, pb_ref, embt_ref, eo_ref,
             fbf_out, f2_out, bm_out, ind_out, diff_out,
             f2x_s, f2_s, e2_s, bm_s, bi_s, acc_s):
    nb = pl.program_id(0)
    kt = pl.program_id(1)
    nbt = pl.num_programs(0)
    nt = f2x_s.shape[0]

    @pl.when(kt == 0)
    def _init():
        f = jnp.dot(z_ref[...], pwt_ref[...],
                    preferred_element_type=jnp.float32) + pb_ref[...]
        f2x = f + f
        f2x_s[...] = f2x
        fbf_out[...] = f2x.astype(jnp.bfloat16)
        f2 = jnp.sum(f * f, axis=1, keepdims=True)
        f2_s[...] = f2
        f2_out[...] = f2
        bm_s[...] = jnp.full((nt, 1), jnp.inf, jnp.float32)
        bi_s[...] = jnp.zeros((nt, 1), jnp.float32)

    @pl.when(nb == 0)
    def _norms():
        eo = eo_ref[...]
        e2c = jnp.sum(eo * eo, axis=1, keepdims=True)
        e2_s[:, pl.ds(kt * _KT, _KT)] = e2c.reshape(1, _KT)

    e = embt_ref[:, pl.ds(kt * _KT, _KT)]
    m2 = jnp.dot(f2x_s[...], e, preferred_element_type=jnp.float32)
    dist = (f2_s[...] - m2) + e2_s[:, pl.ds(kt * _KT, _KT)]

    tmin = jnp.min(dist, axis=1, keepdims=True)
    cols = jax.lax.broadcasted_iota(jnp.int32, dist.shape, 1)
    big = jnp.int32(2147480000)
    idx = jnp.min(jnp.where(dist == tmin, cols, big), axis=1, keepdims=True)
    idx = (idx + kt * _KT).astype(jnp.float32)

    bm_old = bm_s[...]
    bm_s[...] = jnp.minimum(bm_old, tmin)
    flip = tmin < bm_old
    bi_s[...] = jnp.where(flip, idx, bi_s[...])

    @pl.when(kt == kt_last)
    def _fin():
        bm_out[...] = bm_s[...]
        ind_out[...] = bi_s[...].astype(jnp.int32).reshape(ind_out.shape)
        part = jnp.sum(bm_s[...]).reshape(1, 1)
        tot = jnp.where(nb == 0, part, acc_s[...] + part)
        acc_s[...] = tot

        @pl.when(nb == nbt - 1)
        def _done():
            diff_out[...] = tot * jnp.float32(12.5 / n_total)


_KTB = 2048
_NTB = 512


def _qb_body(embt_ref, f_ref, f2_ref, bm_ref, lp_out,
             ebf_s, e2_s, me_s, s_s, c_s):
    nb = pl.program_id(0)
    p = pl.program_id(1)
    kt = pl.program_id(2)
    nt = s_s.shape[0]

    @pl.when((nb == 0) & (p == 0))
    def _stage():
        es = embt_ref[:, pl.ds(kt * _KTB, _KTB)]
        ebf_s[:, pl.ds(kt * _KTB, _KTB)] = es.astype(jnp.bfloat16)
        e2_s[:, pl.ds(kt * _KTB, _KTB)] = jnp.sum(es * es, axis=0,
                                                  keepdims=True)

    @pl.when(p == 0)
    def _accum():
        @pl.when(kt == 0)
        def _z():
            s_s[...] = jnp.zeros((nt, 1), jnp.float32)
        e = ebf_s[:, pl.ds(kt * _KTB, _KTB)]
        m2 = jnp.dot(f_ref[...], e, preferred_element_type=jnp.float32)
        me = m2 - e2_s[:, pl.ds(kt * _KTB, _KTB)]
        me_s[:, pl.ds(kt * _KTB, _KTB)] = me
        cb = bm_ref[...] - f2_ref[...]
        s_s[...] = s_s[...] + jnp.sum(jnp.exp(me + cb), axis=1, keepdims=True)

    @pl.when(p == 1)
    def _write():
        @pl.when(kt == 0)
        def _c():
            c_s[...] = (f2_ref[...] - bm_ref[...]) + jnp.log(s_s[...])
        lp_out[...] = me_s[:, pl.ds(kt * _KTB, _KTB)] - c_s[...]


def _gather_rows(ind2, embed_w):
    nrow, lanes = ind2.shape
    n = nrow * lanes
    k, d = embed_w.shape
    nw = 32
    bpw = n // nw
    chunks = bpw // lanes
    mesh = plsc.VectorSubcoreMesh(core_axis_name="c", subcore_axis_name="s")

    @functools.partial(
        pl.kernel,
        out_type=jax.ShapeDtypeStruct((n, d), jnp.float32),
        mesh=mesh,
        scratch_types=[
            pltpu.VMEM((chunks, lanes), jnp.int32),
            pltpu.VMEM((bpw, d), jnp.float32),
            pltpu.SemaphoreType.DMA,
        ],
    )
    def gk(idx_hbm, tab_hbm, out_hbm, idx_v, rows_v, sem):
        wid = jax.lax.axis_index("s") * 2 + jax.lax.axis_index("c")
        pltpu.sync_copy(idx_hbm.at[pl.ds(wid * chunks, chunks)], idx_v)
        cps = [
            pltpu.async_copy(tab_hbm.at[idx_v.at[j]],
                             rows_v.at[pl.ds(j * lanes, lanes)], sem)
            for j in range(chunks)
        ]
        for cp in cps:
            cp.wait()
        pltpu.sync_copy(rows_v, out_hbm.at[pl.ds(wid * bpw, bpw)])

    return gk(ind2, embed_w)


def kernel(z, proj_w, proj_b, embed_w):
    bz, c, h, w = z.shape
    d = proj_w.shape[0]
    k = embed_w.shape[0]
    n = bz * h * w
    nk = k // _KT
    nb = n // _NT

    z_rows = z.transpose(0, 2, 3, 1).reshape(n, c)
    pwt = proj_w.T
    pb_row = proj_b.reshape(1, d)
    embt = embed_w.T

    fbf, f2col, bmcol, ind64, diffo = pl.pallas_call(
        functools.partial(_qa_body, nk - 1, n * d),
        grid=(nb, nk),
        in_specs=[
            pl.BlockSpec((_NT, c), lambda b, t: (b, 0)),
            pl.BlockSpec((c, d), lambda b, t: (0, 0)),
            pl.BlockSpec((1, d), lambda b, t: (0, 0)),
            pl.BlockSpec((d, k), lambda b, t: (0, 0)),
            pl.BlockSpec((_KT, d), lambda b, t: (t, 0)),
        ],
        out_specs=[
            pl.BlockSpec((_NT, d), lambda b, t: (b, 0)),
            pl.BlockSpec((_NT, 1), lambda b, t: (b, 0)),
            pl.BlockSpec((_NT, 1), lambda b, t: (b, 0)),
            pl.BlockSpec((_NT // 128, 128), lambda b, t: (b, 0)),
            pl.BlockSpec((1, 1), lambda b, t: (0, 0)),
        ],
        out_shape=[
            jax.ShapeDtypeStruct((n, d), jnp.bfloat16),
            jax.ShapeDtypeStruct((n, 1), jnp.float32),
            jax.ShapeDtypeStruct((n, 1), jnp.float32),
            jax.ShapeDtypeStruct((n // 128, 128), jnp.int32),
            jax.ShapeDtypeStruct((1, 1), jnp.float32),
        ],
        scratch_shapes=[
            pltpu.VMEM((_NT, d), jnp.float32),
            pltpu.VMEM((_NT, 1), jnp.float32),
            pltpu.VMEM((1, k), jnp.float32),
            pltpu.VMEM((_NT, 1), jnp.float32),
            pltpu.VMEM((_NT, 1), jnp.float32),
            pltpu.VMEM((1, 1), jnp.float32),
        ],
    )(z_rows, pwt, pb_row, embt, embed_w)

    zq_flat = _gather_rows(ind64, embed_w)

    lp = pl.pallas_call(
        _qb_body,
        grid=(n // _NTB, 2, k // _KTB),
        in_specs=[
            pl.BlockSpec((d, k), lambda b, p, t: (0, 0)),
            pl.BlockSpec((_NTB, d), lambda b, p, t: (b, 0)),
            pl.BlockSpec((_NTB, 1), lambda b, p, t: (b, 0)),
            pl.BlockSpec((_NTB, 1), lambda b, p, t: (b, 0)),
        ],
        out_specs=pl.BlockSpec((_NTB, _KTB), lambda b, p, t: (b, t * p)),
        out_shape=jax.ShapeDtypeStruct((n, k), jnp.float32),
        scratch_shapes=[pltpu.VMEM((d, k), jnp.bfloat16),
                        pltpu.VMEM((1, k), jnp.float32),
                        pltpu.VMEM((_NTB, k), jnp.float32),
                        pltpu.VMEM((_NTB, 1), jnp.float32),
                        pltpu.VMEM((_NTB, 1), jnp.float32)],
    )(embt, fbf, f2col, bmcol)

    z_q = zq_flat.reshape(bz, h, w, d).transpose(0, 3, 1, 2)
    log_priors = lp.reshape(bz, h, w, k).transpose(0, 3, 1, 2)
    ind = ind64.reshape(bz, h, w)
    diff = diffo.reshape(())
    return (z_q, diff, ind, log_priors)


# speedup vs baseline: 1.1400x; 1.0700x over previous
import functools

import jax
import jax.numpy as jnp
from jax.experimental import pallas as pl
from jax.experimental.pallas import tpu as pltpu
from jax.experimental.pallas import tpu_sc as plsc

_KT = 1024
_NT = 1024


def _qa_body(kt_last, n_total, z_ref, pwt_ref, pb_ref, embt_ref, eo_ref,
             fbf_out, f2_out, bm_out, ind_out, diff_out,
             f2x_s, f2_s, e2_s, bm_s, bi_s, acc_s):
    nb = pl.program_id(0)
    kt = pl.program_id(1)
    nbt = pl.num_programs(0)
    nt = f2x_s.shape[0]

    @pl.when(kt == 0)
    def _init():
        f = jnp.dot(z_ref[...], pwt_ref[...],
                    preferred_element_type=jnp.float32) + pb_ref[...]
        f2x = f + f
        f2x_s[...] = f2x
        fbf_out[...] = f2x.astype(jnp.bfloat16)
        f2 = jnp.sum(f * f, axis=1, keepdims=True)
        f2_s[...] = f2
        f2_out[...] = f2
        bm_s[...] = jnp.full((nt, 1), jnp.inf, jnp.float32)
        bi_s[...] = jnp.zeros((nt, 1), jnp.float32)

    @pl.when(nb == 0)
    def _norms():
        eo = eo_ref[...]
        e2c = jnp.sum(eo * eo, axis=1, keepdims=True)
        e2_s[:, pl.ds(kt * _KT, _KT)] = e2c.reshape(1, _KT)

    e = embt_ref[:, pl.ds(kt * _KT, _KT)]
    m2 = jnp.dot(f2x_s[...], e, preferred_element_type=jnp.float32)
    dist = (f2_s[...] - m2) + e2_s[:, pl.ds(kt * _KT, _KT)]

    tmin = jnp.min(dist, axis=1, keepdims=True)
    cols = jax.lax.broadcasted_iota(jnp.int32, dist.shape, 1)
    big = jnp.int32(2147480000)
    idx = jnp.min(jnp.where(dist == tmin, cols, big), axis=1, keepdims=True)
    idx = (idx + kt * _KT).astype(jnp.float32)

    bm_old = bm_s[...]
    bm_s[...] = jnp.minimum(bm_old, tmin)
    flip = tmin < bm_old
    bi_s[...] = jnp.where(flip, idx, bi_s[...])

    @pl.when(kt == kt_last)
    def _fin():
        bm_out[...] = bm_s[...]
        ind_out[...] = bi_s[...].astype(jnp.int32).reshape(ind_out.shape)
        part = jnp.sum(bm_s[...]).reshape(1, 1)
        tot = jnp.where(nb == 0, part, acc_s[...] + part)
        acc_s[...] = tot

        @pl.when(nb == nbt - 1)
        def _done():
            diff_out[...] = tot * jnp.float32(12.5 / n_total)


_KTB = 2048
_NTB = 512


def _qb_body(ebf_ref, f_ref, f2c_ref, bmc_ref, f2p_ref, bmp_ref, lp_out,
             e2_s, me_s, s_s, c_s):
    b = pl.program_id(0)
    t = pl.program_id(1)
    nbt = pl.num_programs(0)
    nt = me_s.shape[1]
    sel = jax.lax.rem(b, 2)
    ksl = pl.ds(t * _KTB, _KTB)

    @pl.when(b == 0)
    def _norms():
        es = ebf_ref[:, ksl].astype(jnp.float32)
        e2_s[:, ksl] = jnp.sum(es * es, axis=0, keepdims=True)

    @pl.when(b < nbt - 1)
    def _p0():
        e = ebf_ref[:, ksl]
        m2 = jnp.dot(f_ref[...], e, preferred_element_type=jnp.float32)
        me = m2 - e2_s[:, ksl]
        me_s[pl.ds(sel, 1), :, ksl] = me[None]
        cb = bmc_ref[...] - f2c_ref[...]
        part = jnp.sum(jnp.exp(me + cb), axis=1, keepdims=True)
        prev = s_s[pl.ds(sel, 1)]
        s_s[pl.ds(sel, 1)] = jnp.where(t == 0, part[None], prev + part[None])

    @pl.when(b > 0)
    def _p1():
        osel = 1 - sel

        @pl.when(t == 0)
        def _c():
            sprev = s_s[pl.ds(osel, 1)].reshape(nt, 1)
            c_s[...] = (f2p_ref[...] - bmp_ref[...]) + jnp.log(sprev)

        lp_out[...] = me_s[pl.ds(osel, 1), :, ksl].reshape(nt, _KTB) - c_s[...]


def _gather_rows(ind2, embed_w):
    nrow, lanes = ind2.shape
    n = nrow * lanes
    k, d = embed_w.shape
    nw = 32
    bpw = n // nw
    chunks = bpw // lanes
    mesh = plsc.VectorSubcoreMesh(core_axis_name="c", subcore_axis_name="s")

    @functools.partial(
        pl.kernel,
        out_type=jax.ShapeDtypeStruct((n, d), jnp.float32),
        mesh=mesh,
        scratch_types=[
            pltpu.VMEM((chunks, lanes), jnp.int32),
            pltpu.VMEM((bpw, d), jnp.float32),
            pltpu.SemaphoreType.DMA,
        ],
    )
    def gk(idx_hbm, tab_hbm, out_hbm, idx_v, rows_v, sem):
        wid = jax.lax.axis_index("s") * 2 + jax.lax.axis_index("c")
        pltpu.sync_copy(idx_hbm.at[pl.ds(wid * chunks, chunks)], idx_v)
        cps = [
            pltpu.async_copy(tab_hbm.at[idx_v.at[j]],
                             rows_v.at[pl.ds(j * lanes, lanes)], sem)
            for j in range(chunks)
        ]
        for cp in cps:
            cp.wait()
        pltpu.sync_copy(rows_v, out_hbm.at[pl.ds(wid * bpw, bpw)])

    return gk(ind2, embed_w)


def kernel(z, proj_w, proj_b, embed_w):
    bz, c, h, w = z.shape
    d = proj_w.shape[0]
    k = embed_w.shape[0]
    n = bz * h * w
    nk = k // _KT
    nb = n // _NT

    z_rows = z.transpose(0, 2, 3, 1).reshape(n, c)
    pwt = proj_w.T
    pb_row = proj_b.reshape(1, d)
    embt = embed_w.T

    fbf, f2col, bmcol, ind64, diffo = pl.pallas_call(
        functools.partial(_qa_body, nk - 1, n * d),
        grid=(nb, nk),
        in_specs=[
            pl.BlockSpec((_NT, c), lambda b, t: (b, 0)),
            pl.BlockSpec((c, d), lambda b, t: (0, 0)),
            pl.BlockSpec((1, d), lambda b, t: (0, 0)),
            pl.BlockSpec((d, k), lambda b, t: (0, 0)),
            pl.BlockSpec((_KT, d), lambda b, t: (t, 0)),
        ],
        out_specs=[
            pl.BlockSpec((_NT, d), lambda b, t: (b, 0)),
            pl.BlockSpec((_NT, 1), lambda b, t: (b, 0)),
            pl.BlockSpec((_NT, 1), lambda b, t: (b, 0)),
            pl.BlockSpec((_NT // 128, 128), lambda b, t: (b, 0)),
            pl.BlockSpec((1, 1), lambda b, t: (0, 0)),
        ],
        out_shape=[
            jax.ShapeDtypeStruct((n, d), jnp.bfloat16),
            jax.ShapeDtypeStruct((n, 1), jnp.float32),
            jax.ShapeDtypeStruct((n, 1), jnp.float32),
            jax.ShapeDtypeStruct((n // 128, 128), jnp.int32),
            jax.ShapeDtypeStruct((1, 1), jnp.float32),
        ],
        scratch_shapes=[
            pltpu.VMEM((_NT, d), jnp.float32),
            pltpu.VMEM((_NT, 1), jnp.float32),
            pltpu.VMEM((1, k), jnp.float32),
            pltpu.VMEM((_NT, 1), jnp.float32),
            pltpu.VMEM((_NT, 1), jnp.float32),
            pltpu.VMEM((1, 1), jnp.float32),
        ],
    )(z_rows, pwt, pb_row, embt, embed_w)

    zq_flat = _gather_rows(ind64, embed_w)

    nbb = n // _NTB
    ebf16 = embt.astype(jnp.bfloat16)
    lp = pl.pallas_call(
        _qb_body,
        grid=(nbb + 1, k // _KTB),
        in_specs=[
            pl.BlockSpec((d, k), lambda b, t: (0, 0)),
            pl.BlockSpec((_NTB, d), lambda b, t: (jnp.minimum(b, nbb - 1), 0)),
            pl.BlockSpec((_NTB, 1), lambda b, t: (jnp.minimum(b, nbb - 1), 0)),
            pl.BlockSpec((_NTB, 1), lambda b, t: (jnp.minimum(b, nbb - 1), 0)),
            pl.BlockSpec((_NTB, 1), lambda b, t: (jnp.maximum(b - 1, 0), 0)),
            pl.BlockSpec((_NTB, 1), lambda b, t: (jnp.maximum(b - 1, 0), 0)),
        ],
        out_specs=pl.BlockSpec((_NTB, _KTB),
                               lambda b, t: (jnp.maximum(b - 1, 0), t)),
        out_shape=jax.ShapeDtypeStruct((n, k), jnp.float32),
        scratch_shapes=[pltpu.VMEM((1, k), jnp.float32),
                        pltpu.VMEM((2, _NTB, k), jnp.float32),
                        pltpu.VMEM((2, _NTB, 1), jnp.float32),
                        pltpu.VMEM((_NTB, 1), jnp.float32)],
    )(ebf16, fbf, f2col, bmcol, f2col, bmcol)

    z_q = zq_flat.reshape(bz, h, w, d).transpose(0, 3, 1, 2)
    log_priors = lp.reshape(bz, h, w, k).transpose(0, 3, 1, 2)
    ind = ind64.reshape(bz, h, w)
    diff = diffo.reshape(())
    return (z_q, diff, ind, log_priors)


# speedup vs baseline: 1.2037x; 1.0559x over previous
import functools

import jax
import jax.numpy as jnp
from jax.experimental import pallas as pl
from jax.experimental.pallas import tpu as pltpu
from jax.experimental.pallas import tpu_sc as plsc

_KT = 2048
_NT = 1024


def _qa_body(kt_last, n_total, z_ref, pwt_ref, pb_ref, embt_ref, eo_ref,
             fbf_out, f2_out, bm_out, ind_out, diff_out,
             f2x_s, f2_s, e2_s, bm_s, bi_s, acc_s):
    nb = pl.program_id(0)
    kt = pl.program_id(1)
    nbt = pl.num_programs(0)
    nt = f2x_s.shape[0]

    @pl.when(kt == 0)
    def _init():
        f = jnp.dot(z_ref[...], pwt_ref[...],
                    preferred_element_type=jnp.float32) + pb_ref[...]
        f2x = f + f
        f2x_s[...] = f2x
        fbf_out[...] = f2x.astype(jnp.bfloat16)
        f2 = jnp.sum(f * f, axis=1, keepdims=True)
        f2_s[...] = f2
        f2_out[...] = f2
        bm_s[...] = jnp.full((nt, 1), jnp.inf, jnp.float32)
        bi_s[...] = jnp.zeros((nt, 1), jnp.float32)

    @pl.when(nb == 0)
    def _norms():
        eo = eo_ref[...]
        e2c = jnp.sum(eo * eo, axis=1, keepdims=True)
        e2_s[:, pl.ds(kt * _KT, _KT)] = e2c.reshape(1, _KT)

    e = embt_ref[:, pl.ds(kt * _KT, _KT)]
    m2 = jnp.dot(f2x_s[...], e, preferred_element_type=jnp.float32)
    dist = (f2_s[...] - m2) + e2_s[:, pl.ds(kt * _KT, _KT)]

    tmin = jnp.min(dist, axis=1, keepdims=True)
    cols = jax.lax.broadcasted_iota(jnp.int32, dist.shape, 1)
    big = jnp.int32(2147480000)
    idx = jnp.min(jnp.where(dist == tmin, cols, big), axis=1, keepdims=True)
    idx = (idx + kt * _KT).astype(jnp.float32)

    bm_old = bm_s[...]
    bm_s[...] = jnp.minimum(bm_old, tmin)
    flip = tmin < bm_old
    bi_s[...] = jnp.where(flip, idx, bi_s[...])

    @pl.when(kt == kt_last)
    def _fin():
        bm_out[...] = bm_s[...]
        ind_out[...] = bi_s[...].astype(jnp.int32).reshape(ind_out.shape)
        part = jnp.sum(bm_s[...]).reshape(1, 1)
        tot = jnp.where(nb == 0, part, acc_s[...] + part)
        acc_s[...] = tot

        @pl.when(nb == nbt - 1)
        def _done():
            diff_out[...] = tot * jnp.float32(12.5 / n_total)


_KTB = 2048
_NTB = 512


def _qb_body(ebf_ref, f_ref, f2c_ref, bmc_ref, f2p_ref, bmp_ref, lp_out,
             e2_s, me_s, s_s, c_s):
    b = pl.program_id(0)
    t = pl.program_id(1)
    nbt = pl.num_programs(0)
    nt = me_s.shape[1]
    sel = jax.lax.rem(b, 2)
    ksl = pl.ds(t * _KTB, _KTB)

    @pl.when(b == 0)
    def _norms():
        es = ebf_ref[:, ksl].astype(jnp.float32)
        e2_s[:, ksl] = jnp.sum(es * es, axis=0, keepdims=True)

    @pl.when(b < nbt - 1)
    def _p0():
        e = ebf_ref[:, ksl]
        m2 = jnp.dot(f_ref[...], e, preferred_element_type=jnp.float32)
        me = m2 - e2_s[:, ksl]
        me_s[pl.ds(sel, 1), :, ksl] = me[None]
        cb = bmc_ref[...] - f2c_ref[...]
        part = jnp.sum(jnp.exp(me + cb), axis=1, keepdims=True)
        prev = s_s[pl.ds(sel, 1)]
        s_s[pl.ds(sel, 1)] = jnp.where(t == 0, part[None], prev + part[None])

    @pl.when(b > 0)
    def _p1():
        osel = 1 - sel

        @pl.when(t == 0)
        def _c():
            sprev = s_s[pl.ds(osel, 1)].reshape(nt, 1)
            c_s[...] = (f2p_ref[...] - bmp_ref[...]) + jnp.log(sprev)

        lp_out[...] = me_s[pl.ds(osel, 1), :, ksl].reshape(nt, _KTB) - c_s[...]


def _gather_rows(ind2, embed_w):
    nrow, lanes = ind2.shape
    n = nrow * lanes
    k, d = embed_w.shape
    nw = 32
    bpw = n // nw
    chunks = bpw // lanes
    mesh = plsc.VectorSubcoreMesh(core_axis_name="c", subcore_axis_name="s")

    @functools.partial(
        pl.kernel,
        out_type=jax.ShapeDtypeStruct((n, d), jnp.float32),
        mesh=mesh,
        scratch_types=[
            pltpu.VMEM((chunks, lanes), jnp.int32),
            pltpu.VMEM((bpw, d), jnp.float32),
            pltpu.SemaphoreType.DMA,
        ],
    )
    def gk(idx_hbm, tab_hbm, out_hbm, idx_v, rows_v, sem):
        wid = jax.lax.axis_index("s") * 2 + jax.lax.axis_index("c")
        pltpu.sync_copy(idx_hbm.at[pl.ds(wid * chunks, chunks)], idx_v)
        cps = [
            pltpu.async_copy(tab_hbm.at[idx_v.at[j]],
                             rows_v.at[pl.ds(j * lanes, lanes)], sem)
            for j in range(chunks)
        ]
        for cp in cps:
            cp.wait()
        pltpu.sync_copy(rows_v, out_hbm.at[pl.ds(wid * bpw, bpw)])

    return gk(ind2, embed_w)


def kernel(z, proj_w, proj_b, embed_w):
    bz, c, h, w = z.shape
    d = proj_w.shape[0]
    k = embed_w.shape[0]
    n = bz * h * w
    nk = k // _KT
    nb = n // _NT

    z_rows = z.transpose(0, 2, 3, 1).reshape(n, c)
    pwt = proj_w.T
    pb_row = proj_b.reshape(1, d)
    embt = embed_w.T

    fbf, f2col, bmcol, ind64, diffo = pl.pallas_call(
        functools.partial(_qa_body, nk - 1, n * d),
        grid=(nb, nk),
        in_specs=[
            pl.BlockSpec((_NT, c), lambda b, t: (b, 0)),
            pl.BlockSpec((c, d), lambda b, t: (0, 0)),
            pl.BlockSpec((1, d), lambda b, t: (0, 0)),
            pl.BlockSpec((d, k), lambda b, t: (0, 0)),
            pl.BlockSpec((_KT, d), lambda b, t: (t, 0)),
        ],
        out_specs=[
            pl.BlockSpec((_NT, d), lambda b, t: (b, 0)),
            pl.BlockSpec((_NT, 1), lambda b, t: (b, 0)),
            pl.BlockSpec((_NT, 1), lambda b, t: (b, 0)),
            pl.BlockSpec((_NT // 128, 128), lambda b, t: (b, 0)),
            pl.BlockSpec((1, 1), lambda b, t: (0, 0)),
        ],
        out_shape=[
            jax.ShapeDtypeStruct((n, d), jnp.bfloat16),
            jax.ShapeDtypeStruct((n, 1), jnp.float32),
            jax.ShapeDtypeStruct((n, 1), jnp.float32),
            jax.ShapeDtypeStruct((n // 128, 128), jnp.int32),
            jax.ShapeDtypeStruct((1, 1), jnp.float32),
        ],
        scratch_shapes=[
            pltpu.VMEM((_NT, d), jnp.float32),
            pltpu.VMEM((_NT, 1), jnp.float32),
            pltpu.VMEM((1, k), jnp.float32),
            pltpu.VMEM((_NT, 1), jnp.float32),
            pltpu.VMEM((_NT, 1), jnp.float32),
            pltpu.VMEM((1, 1), jnp.float32),
        ],
    )(z_rows, pwt, pb_row, embt, embed_w)

    zq_flat = _gather_rows(ind64, embed_w)

    nbb = n // _NTB
    ebf16 = embt.astype(jnp.bfloat16)
    lp = pl.pallas_call(
        _qb_body,
        grid=(nbb + 1, k // _KTB),
        in_specs=[
            pl.BlockSpec((d, k), lambda b, t: (0, 0)),
            pl.BlockSpec((_NTB, d), lambda b, t: (jnp.minimum(b, nbb - 1), 0)),
            pl.BlockSpec((_NTB, 1), lambda b, t: (jnp.minimum(b, nbb - 1), 0)),
            pl.BlockSpec((_NTB, 1), lambda b, t: (jnp.minimum(b, nbb - 1), 0)),
            pl.BlockSpec((_NTB, 1), lambda b, t: (jnp.maximum(b - 1, 0), 0)),
            pl.BlockSpec((_NTB, 1), lambda b, t: (jnp.maximum(b - 1, 0), 0)),
        ],
        out_specs=pl.BlockSpec((_NTB, _KTB),
                               lambda b, t: (jnp.maximum(b - 1, 0), t)),
        out_shape=jax.ShapeDtypeStruct((n, k), jnp.float32),
        scratch_shapes=[pltpu.VMEM((1, k), jnp.float32),
                        pltpu.VMEM((2, _NTB, k), jnp.float32),
                        pltpu.VMEM((2, _NTB, 1), jnp.float32),
                        pltpu.VMEM((_NTB, 1), jnp.float32)],
    )(ebf16, fbf, f2col, bmcol, f2col, bmcol)

    z_q = zq_flat.reshape(bz, h, w, d).transpose(0, 3, 1, 2)
    log_priors = lp.reshape(bz, h, w, k).transpose(0, 3, 1, 2)
    ind = ind64.reshape(bz, h, w)
    diff = diffo.reshape(())
    return (z_q, diff, ind, log_priors)


# speedup vs baseline: 1.2353x; 1.0263x over previous
import functools

import jax
import jax.numpy as jnp
from jax.experimental import pallas as pl
from jax.experimental.pallas import tpu as pltpu
from jax.experimental.pallas import tpu_sc as plsc

_KT = 4096
_NT = 1024


def _qa_body(kt_last, n_total, z_ref, pwt_ref, pb_ref, embt_ref, eo_ref,
             fbf_out, f2_out, bm_out, ind_out, diff_out,
             f2x_s, f2_s, e2_s, bm_s, bi_s, acc_s):
    nb = pl.program_id(0)
    kt = pl.program_id(1)
    nbt = pl.num_programs(0)
    nt = f2x_s.shape[0]

    @pl.when(kt == 0)
    def _init():
        f = jnp.dot(z_ref[...], pwt_ref[...],
                    preferred_element_type=jnp.float32) + pb_ref[...]
        f2x = f + f
        f2x_s[...] = f2x
        fbf_out[...] = f2x.astype(jnp.bfloat16)
        f2 = jnp.sum(f * f, axis=1, keepdims=True)
        f2_s[...] = f2
        f2_out[...] = f2
        bm_s[...] = jnp.full((nt, 1), jnp.inf, jnp.float32)
        bi_s[...] = jnp.zeros((nt, 1), jnp.float32)

    @pl.when(nb == 0)
    def _norms():
        eo = eo_ref[...]
        e2c = jnp.sum(eo * eo, axis=1, keepdims=True)
        e2_s[:, pl.ds(kt * _KT, _KT)] = e2c.reshape(1, _KT)

    e = embt_ref[:, pl.ds(kt * _KT, _KT)]
    m2 = jnp.dot(f2x_s[...], e, preferred_element_type=jnp.float32)
    dist = (f2_s[...] - m2) + e2_s[:, pl.ds(kt * _KT, _KT)]

    tmin = jnp.min(dist, axis=1, keepdims=True)
    cols = jax.lax.broadcasted_iota(jnp.int32, dist.shape, 1)
    big = jnp.int32(2147480000)
    idx = jnp.min(jnp.where(dist == tmin, cols, big), axis=1, keepdims=True)
    idx = (idx + kt * _KT).astype(jnp.float32)

    bm_old = bm_s[...]
    bm_s[...] = jnp.minimum(bm_old, tmin)
    flip = tmin < bm_old
    bi_s[...] = jnp.where(flip, idx, bi_s[...])

    @pl.when(kt == kt_last)
    def _fin():
        bm_out[...] = bm_s[...]
        ind_out[...] = bi_s[...].astype(jnp.int32).reshape(ind_out.shape)
        part = jnp.sum(bm_s[...]).reshape(1, 1)
        tot = jnp.where(nb == 0, part, acc_s[...] + part)
        acc_s[...] = tot

        @pl.when(nb == nbt - 1)
        def _done():
            diff_out[...] = tot * jnp.float32(12.5 / n_total)


_KTB = 2048
_NTB = 512


def _qb_body(ebf_ref, f_ref, f2c_ref, bmc_ref, f2p_ref, bmp_ref, lp_out,
             e2_s, me_s, s_s, c_s):
    b = pl.program_id(0)
    t = pl.program_id(1)
    nbt = pl.num_programs(0)
    nt = me_s.shape[1]
    sel = jax.lax.rem(b, 2)
    ksl = pl.ds(t * _KTB, _KTB)

    @pl.when(b == 0)
    def _norms():
        es = ebf_ref[:, ksl].astype(jnp.float32)
        e2_s[:, ksl] = jnp.sum(es * es, axis=0, keepdims=True)

    @pl.when(b < nbt - 1)
    def _p0():
        e = ebf_ref[:, ksl]
        m2 = jnp.dot(f_ref[...], e, preferred_element_type=jnp.float32)
        me = m2 - e2_s[:, ksl]
        me_s[pl.ds(sel, 1), :, ksl] = me[None]
        cb = bmc_ref[...] - f2c_ref[...]
        part = jnp.sum(jnp.exp(me + cb), axis=1, keepdims=True)
        prev = s_s[pl.ds(sel, 1)]
        s_s[pl.ds(sel, 1)] = jnp.where(t == 0, part[None], prev + part[None])

    @pl.when(b > 0)
    def _p1():
        osel = 1 - sel

        @pl.when(t == 0)
        def _c():
            sprev = s_s[pl.ds(osel, 1)].reshape(nt, 1)
            c_s[...] = (f2p_ref[...] - bmp_ref[...]) + jnp.log(sprev)

        lp_out[...] = me_s[pl.ds(osel, 1), :, ksl].reshape(nt, _KTB) - c_s[...]


def _gather_rows(ind2, embed_w):
    nrow, lanes = ind2.shape
    n = nrow * lanes
    k, d = embed_w.shape
    nw = 32
    bpw = n // nw
    chunks = bpw // lanes
    mesh = plsc.VectorSubcoreMesh(core_axis_name="c", subcore_axis_name="s")

    @functools.partial(
        pl.kernel,
        out_type=jax.ShapeDtypeStruct((n, d), jnp.float32),
        mesh=mesh,
        scratch_types=[
            pltpu.VMEM((chunks, lanes), jnp.int32),
            pltpu.VMEM((bpw, d), jnp.float32),
            pltpu.SemaphoreType.DMA,
        ],
    )
    def gk(idx_hbm, tab_hbm, out_hbm, idx_v, rows_v, sem):
        wid = jax.lax.axis_index("s") * 2 + jax.lax.axis_index("c")
        pltpu.sync_copy(idx_hbm.at[pl.ds(wid * chunks, chunks)], idx_v)
        cps = [
            pltpu.async_copy(tab_hbm.at[idx_v.at[j]],
                             rows_v.at[pl.ds(j * lanes, lanes)], sem)
            for j in range(chunks)
        ]
        for cp in cps:
            cp.wait()
        pltpu.sync_copy(rows_v, out_hbm.at[pl.ds(wid * bpw, bpw)])

    return gk(ind2, embed_w)


def kernel(z, proj_w, proj_b, embed_w):
    bz, c, h, w = z.shape
    d = proj_w.shape[0]
    k = embed_w.shape[0]
    n = bz * h * w
    nk = k // _KT
    nb = n // _NT

    z_rows = z.transpose(0, 2, 3, 1).reshape(n, c)
    pwt = proj_w.T
    pb_row = proj_b.reshape(1, d)
    embt = embed_w.T

    fbf, f2col, bmcol, ind64, diffo = pl.pallas_call(
        functools.partial(_qa_body, nk - 1, n * d),
        grid=(nb, nk),
        in_specs=[
            pl.BlockSpec((_NT, c), lambda b, t: (b, 0)),
            pl.BlockSpec((c, d), lambda b, t: (0, 0)),
            pl.BlockSpec((1, d), lambda b, t: (0, 0)),
            pl.BlockSpec((d, k), lambda b, t: (0, 0)),
            pl.BlockSpec((_KT, d), lambda b, t: (t, 0)),
        ],
        out_specs=[
            pl.BlockSpec((_NT, d), lambda b, t: (b, 0)),
            pl.BlockSpec((_NT, 1), lambda b, t: (b, 0)),
            pl.BlockSpec((_NT, 1), lambda b, t: (b, 0)),
            pl.BlockSpec((_NT // 128, 128), lambda b, t: (b, 0)),
            pl.BlockSpec((1, 1), lambda b, t: (0, 0)),
        ],
        out_shape=[
            jax.ShapeDtypeStruct((n, d), jnp.bfloat16),
            jax.ShapeDtypeStruct((n, 1), jnp.float32),
            jax.ShapeDtypeStruct((n, 1), jnp.float32),
            jax.ShapeDtypeStruct((n // 128, 128), jnp.int32),
            jax.ShapeDtypeStruct((1, 1), jnp.float32),
        ],
        scratch_shapes=[
            pltpu.VMEM((_NT, d), jnp.float32),
            pltpu.VMEM((_NT, 1), jnp.float32),
            pltpu.VMEM((1, k), jnp.float32),
            pltpu.VMEM((_NT, 1), jnp.float32),
            pltpu.VMEM((_NT, 1), jnp.float32),
            pltpu.VMEM((1, 1), jnp.float32),
        ],
    )(z_rows, pwt, pb_row, embt, embed_w)

    zq_flat = _gather_rows(ind64, embed_w)

    nbb = n // _NTB
    ebf16 = embt.astype(jnp.bfloat16)
    lp = pl.pallas_call(
        _qb_body,
        grid=(nbb + 1, k // _KTB),
        in_specs=[
            pl.BlockSpec((d, k), lambda b, t: (0, 0)),
            pl.BlockSpec((_NTB, d), lambda b, t: (jnp.minimum(b, nbb - 1), 0)),
            pl.BlockSpec((_NTB, 1), lambda b, t: (jnp.minimum(b, nbb - 1), 0)),
            pl.BlockSpec((_NTB, 1), lambda b, t: (jnp.minimum(b, nbb - 1), 0)),
            pl.BlockSpec((_NTB, 1), lambda b, t: (jnp.maximum(b - 1, 0), 0)),
            pl.BlockSpec((_NTB, 1), lambda b, t: (jnp.maximum(b - 1, 0), 0)),
        ],
        out_specs=pl.BlockSpec((_NTB, _KTB),
                               lambda b, t: (jnp.maximum(b - 1, 0), t)),
        out_shape=jax.ShapeDtypeStruct((n, k), jnp.float32),
        scratch_shapes=[pltpu.VMEM((1, k), jnp.float32),
                        pltpu.VMEM((2, _NTB, k), jnp.float32),
                        pltpu.VMEM((2, _NTB, 1), jnp.float32),
                        pltpu.VMEM((_NTB, 1), jnp.float32)],
    )(ebf16, fbf, f2col, bmcol, f2col, bmcol)

    z_q = zq_flat.reshape(bz, h, w, d).transpose(0, 3, 1, 2)
    log_priors = lp.reshape(bz, h, w, k).transpose(0, 3, 1, 2)
    ind = ind64.reshape(bz, h, w)
    diff = diffo.reshape(())
    return (z_q, diff, ind, log_priors)


# speedup vs baseline: 1.3174x; 1.0665x over previous
import functools

import jax
import jax.numpy as jnp
from jax.experimental import pallas as pl
from jax.experimental.pallas import tpu as pltpu
from jax.experimental.pallas import tpu_sc as plsc

_KT = 4096
_NT = 1024


def _qa_body(kt_last, n_total, z_ref, pwt_ref, pb_ref, embt_ref, eo_ref,
             fbf_out, f2_out, bm_out, ind_out, diff_out,
             f2x_s, f2_s, e2_s, bm_s, bi_s, acc_s):
    nb = pl.program_id(0)
    kt = pl.program_id(1)
    nbt = pl.num_programs(0)
    nt = f2x_s.shape[0]

    @pl.when(kt == 0)
    def _init():
        f = jnp.dot(z_ref[...], pwt_ref[...],
                    preferred_element_type=jnp.float32) + pb_ref[...]
        f2x = f + f
        f2x_s[...] = f2x
        fbf_out[...] = f2x.astype(jnp.bfloat16)
        f2 = jnp.sum(f * f, axis=1, keepdims=True)
        f2_s[...] = f2
        f2_out[...] = f2
        bm_s[...] = jnp.full((nt, 1), jnp.inf, jnp.float32)
        bi_s[...] = jnp.zeros((nt, 1), jnp.float32)

    @pl.when(nb == 0)
    def _norms():
        eo = eo_ref[...]
        e2c = jnp.sum(eo * eo, axis=1, keepdims=True)
        e2_s[:, pl.ds(kt * _KT, _KT)] = e2c.reshape(1, _KT)

    e = embt_ref[:, pl.ds(kt * _KT, _KT)]
    m2 = jnp.dot(f2x_s[...], e, preferred_element_type=jnp.float32)
    dist = (f2_s[...] - m2) + e2_s[:, pl.ds(kt * _KT, _KT)]

    tmin = jnp.min(dist, axis=1, keepdims=True)
    cols = jax.lax.broadcasted_iota(jnp.int32, dist.shape, 1)
    big = jnp.int32(2147480000)
    idx = jnp.min(jnp.where(dist == tmin, cols, big), axis=1, keepdims=True)
    idx = (idx + kt * _KT).astype(jnp.float32)

    bm_old = bm_s[...]
    bm_s[...] = jnp.minimum(bm_old, tmin)
    flip = tmin < bm_old
    bi_s[...] = jnp.where(flip, idx, bi_s[...])

    @pl.when(kt == kt_last)
    def _fin():
        bm_out[...] = bm_s[...]
        ind_out[...] = bi_s[...].astype(jnp.int32).reshape(ind_out.shape)
        part = jnp.sum(bm_s[...]).reshape(1, 1)
        tot = jnp.where(nb == 0, part, acc_s[...] + part)
        acc_s[...] = tot

        @pl.when(nb == nbt - 1)
        def _done():
            diff_out[...] = tot * jnp.float32(12.5 / n_total)


_KTB = 4096
_NTB = 512


def _qb_body(ebf_ref, f_ref, f2c_ref, bmc_ref, f2p_ref, bmp_ref, lp_out,
             e2_s, me_s, s_s, c_s):
    b = pl.program_id(0)
    t = pl.program_id(1)
    nbt = pl.num_programs(0)
    nt = me_s.shape[1]
    sel = jax.lax.rem(b, 2)
    ksl = pl.ds(t * _KTB, _KTB)

    @pl.when(b == 0)
    def _norms():
        es = ebf_ref[:, ksl].astype(jnp.float32)
        e2_s[:, ksl] = jnp.sum(es * es, axis=0, keepdims=True)

    @pl.when(b < nbt - 1)
    def _p0():
        e = ebf_ref[:, ksl]
        m2 = jnp.dot(f_ref[...], e, preferred_element_type=jnp.float32)
        me = m2 - e2_s[:, ksl]
        me_s[pl.ds(sel, 1), :, ksl] = me[None]
        cb = bmc_ref[...] - f2c_ref[...]
        part = jnp.sum(jnp.exp(me + cb), axis=1, keepdims=True)
        prev = s_s[pl.ds(sel, 1)]
        s_s[pl.ds(sel, 1)] = jnp.where(t == 0, part[None], prev + part[None])

    @pl.when(b > 0)
    def _p1():
        osel = 1 - sel

        @pl.when(t == 0)
        def _c():
            sprev = s_s[pl.ds(osel, 1)].reshape(nt, 1)
            c_s[...] = (f2p_ref[...] - bmp_ref[...]) + jnp.log(sprev)

        lp_out[...] = me_s[pl.ds(osel, 1), :, ksl].reshape(nt, _KTB) - c_s[...]


def _gather_rows(ind2, embed_w):
    nrow, lanes = ind2.shape
    n = nrow * lanes
    k, d = embed_w.shape
    nw = 32
    bpw = n // nw
    chunks = bpw // lanes
    mesh = plsc.VectorSubcoreMesh(core_axis_name="c", subcore_axis_name="s")

    @functools.partial(
        pl.kernel,
        out_type=jax.ShapeDtypeStruct((n, d), jnp.float32),
        mesh=mesh,
        scratch_types=[
            pltpu.VMEM((chunks, lanes), jnp.int32),
            pltpu.VMEM((bpw, d), jnp.float32),
            pltpu.SemaphoreType.DMA,
        ],
    )
    def gk(idx_hbm, tab_hbm, out_hbm, idx_v, rows_v, sem):
        wid = jax.lax.axis_index("s") * 2 + jax.lax.axis_index("c")
        pltpu.sync_copy(idx_hbm.at[pl.ds(wid * chunks, chunks)], idx_v)
        cps = [
            pltpu.async_copy(tab_hbm.at[idx_v.at[j]],
                             rows_v.at[pl.ds(j * lanes, lanes)], sem)
            for j in range(chunks)
        ]
        for cp in cps:
            cp.wait()
        pltpu.sync_copy(rows_v, out_hbm.at[pl.ds(wid * bpw, bpw)])

    return gk(ind2, embed_w)


def kernel(z, proj_w, proj_b, embed_w):
    bz, c, h, w = z.shape
    d = proj_w.shape[0]
    k = embed_w.shape[0]
    n = bz * h * w
    nk = k // _KT
    nb = n // _NT

    z_rows = z.transpose(0, 2, 3, 1).reshape(n, c)
    pwt = proj_w.T
    pb_row = proj_b.reshape(1, d)
    embt = embed_w.T

    fbf, f2col, bmcol, ind64, diffo = pl.pallas_call(
        functools.partial(_qa_body, nk - 1, n * d),
        grid=(nb, nk),
        in_specs=[
            pl.BlockSpec((_NT, c), lambda b, t: (b, 0)),
            pl.BlockSpec((c, d), lambda b, t: (0, 0)),
            pl.BlockSpec((1, d), lambda b, t: (0, 0)),
            pl.BlockSpec((d, k), lambda b, t: (0, 0)),
            pl.BlockSpec((_KT, d), lambda b, t: (t, 0)),
        ],
        out_specs=[
            pl.BlockSpec((_NT, d), lambda b, t: (b, 0)),
            pl.BlockSpec((_NT, 1), lambda b, t: (b, 0)),
            pl.BlockSpec((_NT, 1), lambda b, t: (b, 0)),
            pl.BlockSpec((_NT // 128, 128), lambda b, t: (b, 0)),
            pl.BlockSpec((1, 1), lambda b, t: (0, 0)),
        ],
        out_shape=[
            jax.ShapeDtypeStruct((n, d), jnp.bfloat16),
            jax.ShapeDtypeStruct((n, 1), jnp.float32),
            jax.ShapeDtypeStruct((n, 1), jnp.float32),
            jax.ShapeDtypeStruct((n // 128, 128), jnp.int32),
            jax.ShapeDtypeStruct((1, 1), jnp.float32),
        ],
        scratch_shapes=[
            pltpu.VMEM((_NT, d), jnp.float32),
            pltpu.VMEM((_NT, 1), jnp.float32),
            pltpu.VMEM((1, k), jnp.float32),
            pltpu.VMEM((_NT, 1), jnp.float32),
            pltpu.VMEM((_NT, 1), jnp.float32),
            pltpu.VMEM((1, 1), jnp.float32),
        ],
    )(z_rows, pwt, pb_row, embt, embed_w)

    zq_flat = _gather_rows(ind64, embed_w)

    nbb = n // _NTB
    ebf16 = embt.astype(jnp.bfloat16)
    lp = pl.pallas_call(
        _qb_body,
        grid=(nbb + 1, k // _KTB),
        in_specs=[
            pl.BlockSpec((d, k), lambda b, t: (0, 0)),
            pl.BlockSpec((_NTB, d), lambda b, t: (jnp.minimum(b, nbb - 1), 0)),
            pl.BlockSpec((_NTB, 1), lambda b, t: (jnp.minimum(b, nbb - 1), 0)),
            pl.BlockSpec((_NTB, 1), lambda b, t: (jnp.minimum(b, nbb - 1), 0)),
            pl.BlockSpec((_NTB, 1), lambda b, t: (jnp.maximum(b - 1, 0), 0)),
            pl.BlockSpec((_NTB, 1), lambda b, t: (jnp.maximum(b - 1, 0), 0)),
        ],
        out_specs=pl.BlockSpec((_NTB, _KTB),
                               lambda b, t: (jnp.maximum(b - 1, 0), t)),
        out_shape=jax.ShapeDtypeStruct((n, k), jnp.float32),
        scratch_shapes=[pltpu.VMEM((1, k), jnp.float32),
                        pltpu.VMEM((2, _NTB, k), jnp.float32),
                        pltpu.VMEM((2, _NTB, 1), jnp.float32),
                        pltpu.VMEM((_NTB, 1), jnp.float32)],
    )(ebf16, fbf, f2col, bmcol, f2col, bmcol)

    z_q = zq_flat.reshape(bz, h, w, d).transpose(0, 3, 1, 2)
    log_priors = lp.reshape(bz, h, w, k).transpose(0, 3, 1, 2)
    ind = ind64.reshape(bz, h, w)
    diff = diffo.reshape(())
    return (z_q, diff, ind, log_priors)
